# baseline (speedup 1.0000x reference)
import functools

import jax
import jax.numpy as jnp
from jax import lax
from jax.experimental import pallas as pl
from jax.experimental.pallas import tpu as pltpu
from jax.experimental.pallas import tpu_sc as plsc

N = 10000
E = 320000
D = 128
G = 16

NC = 2
NS = 16
L = 16
W = NC * NS
EPW = E // W
K = 40
NCHUNK = EPW // K
NPAD = 10240
RPT = NPAD // NS


def _linear_body(x_ref, w_ref, b_ref, o_ref):
    o_ref[...] = lax.dot_general(
        x_ref[...], w_ref[...], (((1,), (1,)), ((), ())),
        preferred_element_type=jnp.float32) + b_ref[...]


def _linear(x, w, b, block_rows):
    rows = x.shape[0]
    return pl.pallas_call(
        _linear_body,
        grid=(rows // block_rows,),
        in_specs=[
            pl.BlockSpec((block_rows, D), lambda i: (i, 0)),
            pl.BlockSpec((D, D), lambda i: (0, 0)),
            pl.BlockSpec((1, D), lambda i: (0, 0)),
        ],
        out_specs=pl.BlockSpec((block_rows, D), lambda i: (i, 0)),
        out_shape=jax.ShapeDtypeStruct((rows, D), jnp.float32),
    )(x, w, b.reshape(1, D))


def _make_edge_kernel():
    mesh = plsc.VectorSubcoreMesh(core_axis_name="c", subcore_axis_name="s")

    @functools.partial(
        pl.kernel,
        mesh=mesh,
        out_type=jax.ShapeDtypeStruct((2 * NPAD, D), jnp.float32),
        scratch_types=[
            pltpu.VMEM((4, 2, K), jnp.int32),
            pltpu.VMEM((K, D), jnp.float32),
            pltpu.VMEM((K, D), jnp.float32),
            pltpu.VMEM((K, D), jnp.float32),
            pltpu.VMEM((K, D), jnp.float32),
            pltpu.VMEM((K, D), jnp.float32),
            pltpu.VMEM((K, D), jnp.float32),
            pltpu.VMEM_SHARED((NPAD, D), jnp.float32),
            pltpu.SemaphoreType.DMA,
            pltpu.SemaphoreType.DMA,
            pltpu.SemaphoreType.DMA,
            pltpu.SemaphoreType.DMA,
            pltpu.SemaphoreType.DMA,
            pltpu.SemaphoreType.DMA,
            pltpu.SemaphoreType.DMA,
            pltpu.SemaphoreType.DMA,
        ],
    )
    def edge_kernel(h_hbm, e_hbm, src_hbm, dst_hbm, zeros_hbm, out_hbm,
                    idxb, hr0, hr1, er0, er1, ms0, ms1, agg,
                    isem0, isem1, gsem0, gsem1, esem0, esem1, ssem0, ssem1):
        hr = (hr0, hr1)
        er = (er0, er1)
        ms = (ms0, ms1)
        isem = (isem0, isem1)
        gsem = (gsem0, gsem1)
        esem = (esem0, esem1)
        ssem = (ssem0, ssem1)

        cid = lax.axis_index("c")
        sid = lax.axis_index("s")
        wid = sid * NC + cid

        pltpu.sync_copy(zeros_hbm.at[pl.ds(sid * RPT, RPT)],
                        agg.at[pl.ds(sid * RPT, RPT)])
        plsc.subcore_barrier()

        base0 = wid * EPW

        def idx_fetch(c, j):
            base = base0 + c * K
            pltpu.async_copy(src_hbm.at[pl.ds(base, K)], idxb.at[j, 0],
                             isem[j % 2])
            pltpu.async_copy(dst_hbm.at[pl.ds(base, K)], idxb.at[j, 1],
                             isem[j % 2])

        def idx_wait(j):
            pltpu.make_async_copy(src_hbm.at[pl.ds(0, K)], idxb.at[j, 0],
                                  isem[j % 2]).wait()
            pltpu.make_async_copy(dst_hbm.at[pl.ds(0, K)], idxb.at[j, 1],
                                  isem[j % 2]).wait()

        def issue_ge(c, j):
            s2 = j % 2
            pltpu.async_copy(e_hbm.at[pl.ds(base0 + c * K, K)], er[s2],
                             esem[s2])
            pltpu.async_copy(h_hbm.at[idxb.at[j, 0]], hr[s2], gsem[s2])

        def compute(j):
            s2 = j % 2
            h_v, e_v, m_v = hr[s2], er[s2], ms[s2]

            @plsc.parallel_loop(0, K, unroll=2)
            def row_body(r):
                for c8 in range(D // L):
                    x = h_v[r, pl.ds(c8 * L, L)] + e_v[r, pl.ds(c8 * L, L)]
                    m_v[r, pl.ds(c8 * L, L)] = x / (1.0 + jnp.exp(-x))

        def step(c, j, wait_scatter=True, prefetch=True, issue_next=True):
            s2 = j % 2
            jn, jnn = (j + 1) % 4, (j + 2) % 4
            if wait_scatter:
                pltpu.make_async_copy(ms[s2], agg.at[idxb.at[j, 1]],
                                      ssem[s2]).wait()
            if prefetch:
                idx_fetch(c + 2, jnn)
            if issue_next:
                idx_wait(jn)
                issue_ge(c + 1, jn)
            pltpu.make_async_copy(e_hbm.at[pl.ds(0, K)], er[s2],
                                  esem[s2]).wait()
            pltpu.make_async_copy(h_hbm.at[idxb.at[j, 0]], hr[s2],
                                  gsem[s2]).wait()
            compute(j)
            pltpu.async_copy(ms[s2], agg.at[idxb.at[j, 1]], ssem[s2],
                             add=True)

        pltpu.sync_copy(src_hbm.at[pl.ds(base0, K)], idxb.at[0, 0])
        pltpu.sync_copy(dst_hbm.at[pl.ds(base0, K)], idxb.at[0, 1])
        idx_fetch(1, 1)
        issue_ge(0, 0)
        step(0, 0, wait_scatter=False)
        step(1, 1, wait_scatter=False)
        step(2, 2)
        step(3, 3)

        def quad(q, carry):
            c = q * 4
            step(c, 0)
            step(c + 1, 1)
            step(c + 2, 2)
            step(c + 3, 3)
            return carry

        lax.fori_loop(1, NCHUNK // 4, quad, 0)

        step(NCHUNK - 2, 0, prefetch=False)
        step(NCHUNK - 1, 1, prefetch=False, issue_next=False)
        pltpu.make_async_copy(ms0, agg.at[idxb.at[0, 1]], ssem0).wait()
        pltpu.make_async_copy(ms1, agg.at[idxb.at[1, 1]], ssem1).wait()

        plsc.subcore_barrier()
        pltpu.sync_copy(agg.at[pl.ds(sid * RPT, RPT)],
                        out_hbm.at[pl.ds(cid * NPAD + sid * RPT, RPT)])

    return edge_kernel


_edge_kernel = _make_edge_kernel()


def _final_body(agg2_ref, h_ref, x0_ref, batch_ref, eps_ref, scale_ref,
                wgt_ref, bias_ref, o_ref):
    agg = agg2_ref[pl.ds(0, N), :] + agg2_ref[pl.ds(NPAD, N), :]
    h = h_ref[...]
    hu = agg + (1.0 + eps_ref[0, 0]) * h
    hu = hu * jax.nn.sigmoid(hu)
    y = hu + x0_ref[...]

    onehot = (batch_ref[...] ==
              lax.broadcasted_iota(jnp.int32, (1, G), 1)).astype(jnp.float32)
    counts = jnp.maximum(jnp.sum(onehot, axis=0), 1.0)
    sums = lax.dot_general(onehot, y, (((0,), (0,)), ((), ())),
                           preferred_element_type=jnp.float32)
    mean = sums / counts[:, None]
    mean_rows = lax.dot_general(onehot, mean, (((1,), (0,)), ((), ())),
                                preferred_element_type=jnp.float32)
    out = y - mean_rows * scale_ref[...]
    vsums = lax.dot_general(onehot, out * out, (((0,), (0,)), ((), ())),
                            preferred_element_type=jnp.float32)
    var = vsums / counts[:, None]
    rstd = lax.rsqrt(var + 1e-5)
    rstd_rows = lax.dot_general(onehot, rstd, (((1,), (0,)), ((), ())),
                                preferred_element_type=jnp.float32)
    o_ref[...] = wgt_ref[...] * out * rstd_rows + bias_ref[...]


def _final(agg2, h, node_h, batch, eps, scale, wgt, bias):
    return pl.pallas_call(
        _final_body,
        out_shape=jax.ShapeDtypeStruct((N, D), jnp.float32),
    )(agg2, h, node_h, batch.reshape(N, 1), eps.reshape(1, 1),
      scale.reshape(1, D), wgt.reshape(1, D), bias.reshape(1, D))


def kernel(node_h, edge_attr, batch, edge_index, W1, b1, Wl, bl, eps_gine,
           gn_weight, gn_bias, gn_mean_scale):
    h = _linear(node_h, W1, b1, 2000)
    e = _linear(edge_attr, Wl, bl, 2000)
    zeros = jnp.zeros((NPAD, D), jnp.float32)
    agg2 = _edge_kernel(h, e, edge_index[0], edge_index[1], zeros)
    return _final(agg2, h, node_h, batch, eps_gine, gn_mean_scale,
                  gn_weight, gn_bias)


# speedup vs baseline: 1.3491x; 1.3491x over previous
import functools

import jax
import jax.numpy as jnp
from jax import lax
from jax.experimental import pallas as pl
from jax.experimental.pallas import tpu as pltpu
from jax.experimental.pallas import tpu_sc as plsc

N = 10000
E = 320000
D = 128
G = 16

NC = 2
NS = 16
L = 16
W = NC * NS
EPW = E // W
K = 40
NCHUNK = EPW // K
NPAD = 10240
RPT = NPAD // NS


def _linear_body(x_ref, w_ref, b_ref, o_ref):
    o_ref[...] = lax.dot_general(
        x_ref[...], w_ref[...], (((1,), (1,)), ((), ())),
        preferred_element_type=jnp.float32) + b_ref[...]


def _linear(x, w, b, block_rows):
    rows = x.shape[0]
    return pl.pallas_call(
        _linear_body,
        grid=(rows // block_rows,),
        in_specs=[
            pl.BlockSpec((block_rows, D), lambda i: (i, 0)),
            pl.BlockSpec((D, D), lambda i: (0, 0)),
            pl.BlockSpec((1, D), lambda i: (0, 0)),
        ],
        out_specs=pl.BlockSpec((block_rows, D), lambda i: (i, 0)),
        out_shape=jax.ShapeDtypeStruct((rows, D), jnp.float32),
    )(x, w, b.reshape(1, D))


def _make_edge_kernel():
    mesh = plsc.VectorSubcoreMesh(core_axis_name="c", subcore_axis_name="s")

    @functools.partial(
        pl.kernel,
        mesh=mesh,
        out_type=jax.ShapeDtypeStruct((2 * NPAD, D), jnp.float32),
        scratch_types=[
            pltpu.VMEM((4, 2, K), jnp.int32),
            pltpu.VMEM((K, D), jnp.float32),
            pltpu.VMEM((K, D), jnp.float32),
            pltpu.VMEM((K, D), jnp.float32),
            pltpu.VMEM((K, D), jnp.float32),
            pltpu.VMEM((K, D), jnp.float32),
            pltpu.VMEM((K, D), jnp.float32),
            pltpu.VMEM_SHARED((NPAD, D), jnp.float32),
            pltpu.SemaphoreType.DMA,
            pltpu.SemaphoreType.DMA,
            pltpu.SemaphoreType.DMA,
            pltpu.SemaphoreType.DMA,
            pltpu.SemaphoreType.DMA,
            pltpu.SemaphoreType.DMA,
            pltpu.SemaphoreType.DMA,
            pltpu.SemaphoreType.DMA,
        ],
    )
    def edge_kernel(h_hbm, e_hbm, src_hbm, dst_hbm, zeros_hbm, out_hbm,
                    idxb, hr0, hr1, er0, er1, ms0, ms1, agg,
                    isem0, isem1, gsem0, gsem1, esem0, esem1, ssem0, ssem1):
        hr = (hr0, hr1)
        er = (er0, er1)
        ms = (ms0, ms1)
        isem = (isem0, isem1)
        gsem = (gsem0, gsem1)
        esem = (esem0, esem1)
        ssem = (ssem0, ssem1)

        cid = lax.axis_index("c")
        sid = lax.axis_index("s")
        wid = sid * NC + cid

        pltpu.sync_copy(zeros_hbm.at[pl.ds(sid * RPT, RPT)],
                        agg.at[pl.ds(sid * RPT, RPT)])
        plsc.subcore_barrier()

        base0 = wid * EPW

        def idx_fetch(c, j):
            base = base0 + c * K
            pltpu.async_copy(src_hbm.at[pl.ds(base, K)], idxb.at[j, 0],
                             isem[j % 2])
            pltpu.async_copy(dst_hbm.at[pl.ds(base, K)], idxb.at[j, 1],
                             isem[j % 2])

        def idx_wait(j):
            pltpu.make_async_copy(src_hbm.at[pl.ds(0, K)], idxb.at[j, 0],
                                  isem[j % 2]).wait()
            pltpu.make_async_copy(dst_hbm.at[pl.ds(0, K)], idxb.at[j, 1],
                                  isem[j % 2]).wait()

        def issue_ge(c, j):
            s2 = j % 2
            pltpu.async_copy(e_hbm.at[pl.ds(base0 + c * K, K)], er[s2],
                             esem[s2])
            pltpu.async_copy(h_hbm.at[idxb.at[j, 0]], hr[s2], gsem[s2])

        def compute(j):
            s2 = j % 2
            h_v, e_v, m_v = hr[s2], er[s2], ms[s2]

            def row_body(r2, carry):
                for dr in range(2):
                    r = r2 * 2 + dr
                    for c8 in range(D // L):
                        x = (h_v[r, pl.ds(c8 * L, L)] +
                             e_v[r, pl.ds(c8 * L, L)])
                        m_v[r, pl.ds(c8 * L, L)] = x / (1.0 + jnp.exp(-x))
                return carry

            lax.fori_loop(0, K // 2, row_body, 0)

        def step(c, j, wait_scatter=True, prefetch=True, issue_next=True):
            s2 = j % 2
            jn, jnn = (j + 1) % 4, (j + 2) % 4
            if wait_scatter:
                pltpu.make_async_copy(ms[s2], agg.at[idxb.at[j, 1]],
                                      ssem[s2]).wait()
            if prefetch:
                idx_fetch(c + 2, jnn)
            if issue_next:
                idx_wait(jn)
                issue_ge(c + 1, jn)
            pltpu.make_async_copy(e_hbm.at[pl.ds(0, K)], er[s2],
                                  esem[s2]).wait()
            pltpu.make_async_copy(h_hbm.at[idxb.at[j, 0]], hr[s2],
                                  gsem[s2]).wait()
            compute(j)
            pltpu.async_copy(ms[s2], agg.at[idxb.at[j, 1]], ssem[s2],
                             add=True)

        pltpu.sync_copy(src_hbm.at[pl.ds(base0, K)], idxb.at[0, 0])
        pltpu.sync_copy(dst_hbm.at[pl.ds(base0, K)], idxb.at[0, 1])
        idx_fetch(1, 1)
        issue_ge(0, 0)
        step(0, 0, wait_scatter=False)
        step(1, 1, wait_scatter=False)
        step(2, 2)
        step(3, 3)

        def quad(q, carry):
            c = q * 4
            step(c, 0)
            step(c + 1, 1)
            step(c + 2, 2)
            step(c + 3, 3)
            return carry

        lax.fori_loop(1, NCHUNK // 4, quad, 0)

        step(NCHUNK - 2, 0, prefetch=False)
        step(NCHUNK - 1, 1, prefetch=False, issue_next=False)
        pltpu.make_async_copy(ms0, agg.at[idxb.at[0, 1]], ssem0).wait()
        pltpu.make_async_copy(ms1, agg.at[idxb.at[1, 1]], ssem1).wait()

        plsc.subcore_barrier()
        pltpu.sync_copy(agg.at[pl.ds(sid * RPT, RPT)],
                        out_hbm.at[pl.ds(cid * NPAD + sid * RPT, RPT)])

    return edge_kernel


_edge_kernel = _make_edge_kernel()


def _final_body(agg2_ref, h_ref, x0_ref, batch_ref, eps_ref, scale_ref,
                wgt_ref, bias_ref, o_ref):
    agg = agg2_ref[pl.ds(0, N), :] + agg2_ref[pl.ds(NPAD, N), :]
    h = h_ref[...]
    hu = agg + (1.0 + eps_ref[0, 0]) * h
    hu = hu * jax.nn.sigmoid(hu)
    y = hu + x0_ref[...]

    onehot = (batch_ref[...] ==
              lax.broadcasted_iota(jnp.int32, (1, G), 1)).astype(jnp.float32)
    counts = jnp.maximum(jnp.sum(onehot, axis=0), 1.0)
    sums = lax.dot_general(onehot, y, (((0,), (0,)), ((), ())),
                           preferred_element_type=jnp.float32)
    mean = sums / counts[:, None]
    mean_rows = lax.dot_general(onehot, mean, (((1,), (0,)), ((), ())),
                                preferred_element_type=jnp.float32)
    out = y - mean_rows * scale_ref[...]
    vsums = lax.dot_general(onehot, out * out, (((0,), (0,)), ((), ())),
                            preferred_element_type=jnp.float32)
    var = vsums / counts[:, None]
    rstd = lax.rsqrt(var + 1e-5)
    rstd_rows = lax.dot_general(onehot, rstd, (((1,), (0,)), ((), ())),
                                preferred_element_type=jnp.float32)
    o_ref[...] = wgt_ref[...] * out * rstd_rows + bias_ref[...]


def _final(agg2, h, node_h, batch, eps, scale, wgt, bias):
    return pl.pallas_call(
        _final_body,
        out_shape=jax.ShapeDtypeStruct((N, D), jnp.float32),
    )(agg2, h, node_h, batch.reshape(N, 1), eps.reshape(1, 1),
      scale.reshape(1, D), wgt.reshape(1, D), bias.reshape(1, D))


def kernel(node_h, edge_attr, batch, edge_index, W1, b1, Wl, bl, eps_gine,
           gn_weight, gn_bias, gn_mean_scale):
    h = _linear(node_h, W1, b1, 2000)
    e = _linear(edge_attr, Wl, bl, 2000)
    zeros = jnp.zeros((NPAD, D), jnp.float32)
    agg2 = _edge_kernel(h, e, edge_index[0], edge_index[1], zeros)
    return _final(agg2, h, node_h, batch, eps_gine, gn_mean_scale,
                  gn_weight, gn_bias)


# speedup vs baseline: 1.4318x; 1.0613x over previous
import functools

import jax
import jax.numpy as jnp
from jax import lax
from jax.experimental import pallas as pl
from jax.experimental.pallas import tpu as pltpu
from jax.experimental.pallas import tpu_sc as plsc

N = 10000
E = 320000
D = 128
G = 16

NC = 2
NS = 16
L = 16
W = NC * NS
EPW = E // W
K = 40
NCHUNK = EPW // K
NPAD = 10240
RPT = NPAD // NS


def _linear_body(x_ref, w_ref, b_ref, o_ref):
    o_ref[...] = lax.dot_general(
        x_ref[...], w_ref[...], (((1,), (1,)), ((), ())),
        preferred_element_type=jnp.float32) + b_ref[...]


def _linear(x, w, b, block_rows):
    rows = x.shape[0]
    return pl.pallas_call(
        _linear_body,
        grid=(rows // block_rows,),
        in_specs=[
            pl.BlockSpec((block_rows, D), lambda i: (i, 0)),
            pl.BlockSpec((D, D), lambda i: (0, 0)),
            pl.BlockSpec((1, D), lambda i: (0, 0)),
        ],
        out_specs=pl.BlockSpec((block_rows, D), lambda i: (i, 0)),
        out_shape=jax.ShapeDtypeStruct((rows, D), jnp.float32),
    )(x, w, b.reshape(1, D))


def _make_edge_kernel():
    mesh = plsc.VectorSubcoreMesh(core_axis_name="c", subcore_axis_name="s")

    @functools.partial(
        pl.kernel,
        mesh=mesh,
        out_type=jax.ShapeDtypeStruct((2 * NPAD, D), jnp.float32),
        scratch_types=[
            pltpu.VMEM((4, 2, K), jnp.int32),
            pltpu.VMEM((K, D), jnp.float32),
            pltpu.VMEM((K, D), jnp.float32),
            pltpu.VMEM((K, D), jnp.float32),
            pltpu.VMEM((K, D), jnp.float32),
            pltpu.VMEM((K, D), jnp.float32),
            pltpu.VMEM((K, D), jnp.float32),
            pltpu.VMEM_SHARED((NPAD, D), jnp.float32),
            pltpu.SemaphoreType.DMA,
            pltpu.SemaphoreType.DMA,
            pltpu.SemaphoreType.DMA,
            pltpu.SemaphoreType.DMA,
            pltpu.SemaphoreType.DMA,
            pltpu.SemaphoreType.DMA,
            pltpu.SemaphoreType.DMA,
            pltpu.SemaphoreType.DMA,
        ],
    )
    def edge_kernel(h_hbm, e_hbm, ei_hbm, out_hbm,
                    idxb, hr0, hr1, er0, er1, ms0, ms1, agg,
                    isem0, isem1, gsem0, gsem1, esem0, esem1, ssem0, ssem1):
        hr = (hr0, hr1)
        er = (er0, er1)
        ms = (ms0, ms1)
        isem = (isem0, isem1)
        gsem = (gsem0, gsem1)
        esem = (esem0, esem1)
        ssem = (ssem0, ssem1)

        cid = lax.axis_index("c")
        sid = lax.axis_index("s")
        wid = sid * NC + cid

        def zrow(r, carry):
            for c8 in range(D // L):
                ms0[r, pl.ds(c8 * L, L)] = jnp.zeros((L,), jnp.float32)
            return carry

        lax.fori_loop(0, K, zrow, 0)
        for i in range(RPT // K):
            pltpu.sync_copy(ms0, agg.at[pl.ds(sid * RPT + i * K, K)])
        plsc.subcore_barrier()

        base0 = wid * EPW

        def idx_fetch(c, j):
            base = base0 + c * K
            pltpu.async_copy(ei_hbm.at[pl.ds(base, K)], idxb.at[j, 0],
                             isem[j % 2])
            pltpu.async_copy(ei_hbm.at[pl.ds(E + base, K)], idxb.at[j, 1],
                             isem[j % 2])

        def idx_wait(j):
            pltpu.make_async_copy(ei_hbm.at[pl.ds(0, K)], idxb.at[j, 0],
                                  isem[j % 2]).wait()
            pltpu.make_async_copy(ei_hbm.at[pl.ds(0, K)], idxb.at[j, 1],
                                  isem[j % 2]).wait()

        def issue_ge(c, j):
            s2 = j % 2
            pltpu.async_copy(e_hbm.at[pl.ds(base0 + c * K, K)], er[s2],
                             esem[s2])
            pltpu.async_copy(h_hbm.at[idxb.at[j, 0]], hr[s2], gsem[s2])

        def compute(j):
            s2 = j % 2
            h_v, e_v, m_v = hr[s2], er[s2], ms[s2]

            def row_body(r2, carry):
                for dr in range(2):
                    r = r2 * 2 + dr
                    for c8 in range(D // L):
                        x = (h_v[r, pl.ds(c8 * L, L)] +
                             e_v[r, pl.ds(c8 * L, L)])
                        m_v[r, pl.ds(c8 * L, L)] = x / (1.0 +
                                                        jnp.exp(x * -1.0))
                return carry

            lax.fori_loop(0, K // 2, row_body, 0)

        def step(c, j, wait_scatter=True, prefetch=True, issue_next=True):
            s2 = j % 2
            jn, jnn = (j + 1) % 4, (j + 2) % 4
            if wait_scatter:
                pltpu.make_async_copy(ms[s2], agg.at[idxb.at[j, 1]],
                                      ssem[s2]).wait()
            if prefetch:
                idx_fetch(c + 2, jnn)
            if issue_next:
                idx_wait(jn)
                issue_ge(c + 1, jn)
            pltpu.make_async_copy(e_hbm.at[pl.ds(0, K)], er[s2],
                                  esem[s2]).wait()
            pltpu.make_async_copy(h_hbm.at[idxb.at[j, 0]], hr[s2],
                                  gsem[s2]).wait()
            compute(j)
            pltpu.async_copy(ms[s2], agg.at[idxb.at[j, 1]], ssem[s2],
                             add=True)

        pltpu.sync_copy(ei_hbm.at[pl.ds(base0, K)], idxb.at[0, 0])
        pltpu.sync_copy(ei_hbm.at[pl.ds(E + base0, K)], idxb.at[0, 1])
        idx_fetch(1, 1)
        issue_ge(0, 0)
        step(0, 0, wait_scatter=False)
        step(1, 1, wait_scatter=False)
        step(2, 2)
        step(3, 3)

        def quad(q, carry):
            c = q * 4
            step(c, 0)
            step(c + 1, 1)
            step(c + 2, 2)
            step(c + 3, 3)
            return carry

        lax.fori_loop(1, NCHUNK // 4, quad, 0)

        step(NCHUNK - 2, 0, prefetch=False)
        step(NCHUNK - 1, 1, prefetch=False, issue_next=False)
        pltpu.make_async_copy(ms0, agg.at[idxb.at[0, 1]], ssem0).wait()
        pltpu.make_async_copy(ms1, agg.at[idxb.at[1, 1]], ssem1).wait()

        plsc.subcore_barrier()
        pltpu.sync_copy(agg.at[pl.ds(sid * RPT, RPT)],
                        out_hbm.at[pl.ds(cid * NPAD + sid * RPT, RPT)])

    return edge_kernel


_edge_kernel = _make_edge_kernel()


def _final_body(agg2_ref, h_ref, x0_ref, batch_ref, eps_ref, scale_ref,
                wgt_ref, bias_ref, o_ref):
    agg = agg2_ref[pl.ds(0, N), :] + agg2_ref[pl.ds(NPAD, N), :]
    h = h_ref[...]
    hu = agg + (1.0 + eps_ref[0, 0]) * h
    hu = hu * jax.nn.sigmoid(hu)
    y = hu + x0_ref[...]

    onehot = (batch_ref[...] ==
              lax.broadcasted_iota(jnp.int32, (1, G), 1)).astype(jnp.float32)
    counts = jnp.maximum(jnp.sum(onehot, axis=0), 1.0)
    sums = lax.dot_general(onehot, y, (((0,), (0,)), ((), ())),
                           preferred_element_type=jnp.float32)
    mean = sums / counts[:, None]
    mean_rows = lax.dot_general(onehot, mean, (((1,), (0,)), ((), ())),
                                preferred_element_type=jnp.float32)
    out = y - mean_rows * scale_ref[...]
    vsums = lax.dot_general(onehot, out * out, (((0,), (0,)), ((), ())),
                            preferred_element_type=jnp.float32)
    var = vsums / counts[:, None]
    rstd = lax.rsqrt(var + 1e-5)
    rstd_rows = lax.dot_general(onehot, rstd, (((1,), (0,)), ((), ())),
                                preferred_element_type=jnp.float32)
    o_ref[...] = wgt_ref[...] * out * rstd_rows + bias_ref[...]


def _final(agg2, h, node_h, batch, eps, scale, wgt, bias):
    return pl.pallas_call(
        _final_body,
        out_shape=jax.ShapeDtypeStruct((N, D), jnp.float32),
    )(agg2, h, node_h, batch.reshape(N, 1), eps.reshape(1, 1),
      scale.reshape(1, D), wgt.reshape(1, D), bias.reshape(1, D))


def kernel(node_h, edge_attr, batch, edge_index, W1, b1, Wl, bl, eps_gine,
           gn_weight, gn_bias, gn_mean_scale):
    h = _linear(node_h, W1, b1, 2000)
    e = _linear(edge_attr, Wl, bl, 2000)
    agg2 = _edge_kernel(h, e, edge_index.reshape(2 * E))
    return _final(agg2, h, node_h, batch, eps_gine, gn_mean_scale,
                  gn_weight, gn_bias)
